# baseline (speedup 1.0000x reference)
import functools
import math

import jax
import jax.numpy as jnp
from jax import lax
from jax.experimental import pallas as pl
from jax.experimental.pallas import tpu as pltpu
from jax.experimental.pallas import tpu_sc as plsc

N = 10000
E = 320000
D = 128
POS_VOC = 1024

NC, NS = 2, 16
NW = NC * NS

CHUNK = 64
CH_PER_W = 160
E_PER_W = CHUNK * CH_PER_W
E_PAD = NW * E_PER_W
ROWS_PAD = 10240
ROWS_PER_TILE = ROWS_PAD // NS
DUMP_ROWS = ROWS_PAD - N

NBUF = 5
SLACK = 2
NPHASE = 10
CH_PER_PH = CH_PER_W // NPHASE

POS_CHUNK = 64
POS_CH_PER_W = 5
POS_PER_W = POS_CHUNK * POS_CH_PER_W
POS_PAD = NW * POS_PER_W


@functools.cache
def _sc_mesh():
    return plsc.VectorSubcoreMesh(
        core_axis_name="c", subcore_axis_name="s", num_cores=NC, num_subcores=NS)


@functools.cache
def _pos_gather_kernel():
    @functools.partial(
        pl.kernel,
        out_type=jax.ShapeDtypeStruct((POS_PAD, D), jnp.float32),
        mesh=_sc_mesh(),
        scratch_types=[
            pltpu.VMEM((POS_CH_PER_W, POS_CHUNK), jnp.int32),
            pltpu.VMEM((POS_CHUNK, D), jnp.float32),
            pltpu.SemaphoreType.DMA,
        ],
    )
    def _pos_gather(table_hbm, pos_hbm, out_hbm, idx_v, rows_v, sem):
        wid = lax.axis_index("c") * NS + lax.axis_index("s")
        pltpu.sync_copy(pos_hbm.at[wid], idx_v)
        for j in range(POS_CH_PER_W):
            pltpu.async_copy(table_hbm.at[idx_v.at[j]], rows_v, sem).wait()
            pltpu.sync_copy(
                rows_v, out_hbm.at[pl.ds(wid * POS_PER_W + j * POS_CHUNK, POS_CHUNK)])

    return _pos_gather


@functools.cache
def _seg_sum_kernel():
    @functools.partial(
        pl.kernel,
        out_type=jax.ShapeDtypeStruct((NC, ROWS_PAD, D), jnp.float32),
        mesh=_sc_mesh(),
        scratch_types=[
            pltpu.VMEM((CH_PER_PH, CHUNK), jnp.int32),
            pltpu.VMEM((CH_PER_PH, CHUNK), jnp.int32),
            pltpu.VMEM((NBUF, CHUNK, D), jnp.float32),
            pltpu.VMEM_SHARED((ROWS_PAD, D), jnp.float32),
        ] + [pltpu.SemaphoreType.DMA] * (2 * NBUF),
    )
    def _seg_sum(h_hbm, src_hbm, dst_hbm, zeros_hbm, out_hbm,
                 idx_s_v, idx_d_v, rows_v, acc_sh, *sems):
        gsems, ssems = sems[:NBUF], sems[NBUF:]
        cid = lax.axis_index("c")
        sid = lax.axis_index("s")
        wid = cid * NS + sid

        def start_gather(b, cj):
            pltpu.async_copy(h_hbm.at[idx_s_v.at[cj]], rows_v.at[b], gsems[b])

        def wait_gather(b):
            pltpu.make_async_copy(
                h_hbm.at[idx_s_v.at[0]], rows_v.at[b], gsems[b]).wait()

        def start_scatter(b, cj):
            pltpu.async_copy(rows_v.at[b], acc_sh.at[idx_d_v.at[cj]],
                             ssems[b], add=True)

        def wait_scatter(b):
            pltpu.make_async_copy(
                rows_v.at[b], acc_sh.at[idx_d_v.at[0]], ssems[b]).wait()

        for ph in range(NPHASE):
            pltpu.sync_copy(src_hbm.at[wid, pl.ds(ph * CH_PER_PH, CH_PER_PH)], idx_s_v)
            pltpu.sync_copy(dst_hbm.at[wid, pl.ds(ph * CH_PER_PH, CH_PER_PH)], idx_d_v)

            for b in range(NBUF):
                start_gather(b, b)
            if ph == 0:
                pltpu.sync_copy(
                    zeros_hbm, acc_sh.at[pl.ds(sid * ROWS_PER_TILE, ROWS_PER_TILE)])
                plsc.subcore_barrier()
            for c in range(CH_PER_PH):
                wait_gather(c % NBUF)
                start_scatter(c % NBUF, c)
                if c >= SLACK and c - SLACK + NBUF < CH_PER_PH:
                    wait_scatter((c - SLACK) % NBUF)
                    start_gather((c - SLACK) % NBUF, c - SLACK + NBUF)
            for c in range(CH_PER_PH - NBUF, CH_PER_PH):
                wait_scatter(c % NBUF)
        plsc.subcore_barrier()
        pltpu.sync_copy(acc_sh.at[pl.ds(sid * ROWS_PER_TILE, ROWS_PER_TILE)],
                        out_hbm.at[cid, pl.ds(sid * ROWS_PER_TILE, ROWS_PER_TILE)])

    return _seg_sum


def _ln(x, g, b):
    m = jnp.mean(x, axis=1, keepdims=True)
    v = jnp.mean((x - m) ** 2, axis=1, keepdims=True)
    return (x - m) * lax.rsqrt(v + 1e-5) * g + b


def _embed_body(ne_ref, pe_ref, g_ref, b_ref, o_ref):
    x = ne_ref[...] * math.sqrt(D) + pe_ref[...]
    o_ref[...] = _ln(x, g_ref[...], b_ref[...])


def _matT(x, w_ref):
    return lax.dot_general(x, w_ref[...], (((1,), (1,)), ((), ())),
                           preferred_element_type=jnp.float32)


def _layer_body(p_ref, h_ref, wl_ref, bl_ref, wr_ref, g_ref, b_ref, o_ref):
    h = h_ref[...]
    agg = p_ref[0] + p_ref[1]
    z = _matT(agg, wl_ref) + bl_ref[...] + _matT(h, wr_ref)
    z = jnp.maximum(z, 0.0) + h
    o_ref[...] = _ln(z, g_ref[...], b_ref[...])


_ROW_BLK = 1000


def _row_spec():
    return pl.BlockSpec((_ROW_BLK, D), lambda i: (i, 0))


def _full_spec(r):
    return pl.BlockSpec((r, D), lambda i: (0, 0))


_embed_call = pl.pallas_call(
    _embed_body,
    grid=(N // _ROW_BLK,),
    in_specs=[_row_spec(), _row_spec(), _full_spec(1), _full_spec(1)],
    out_specs=_row_spec(),
    out_shape=jax.ShapeDtypeStruct((N, D), jnp.float32),
)

_layer_call = pl.pallas_call(
    _layer_body,
    grid=(N // _ROW_BLK,),
    in_specs=[pl.BlockSpec((NC, _ROW_BLK, D), lambda i: (0, i, 0)),
              _row_spec(),
              _full_spec(D), _full_spec(1), _full_spec(D),
              _full_spec(1), _full_spec(1)],
    out_specs=_row_spec(),
    out_shape=jax.ShapeDtypeStruct((N, D), jnp.float32),
)


def kernel(node_emb, pos, edge, pos_table, emb_ln_g, emb_ln_b,
           Wl0, bl0, Wr0, ln0_g, ln0_b, Wl1, bl1, Wr1, ln1_g, ln1_b):
    f32 = jnp.float32
    pos_p = jnp.concatenate(
        [pos.astype(jnp.int32), jnp.arange(POS_PAD - N, dtype=jnp.int32) % POS_VOC]
    ).reshape(NW, POS_CH_PER_W, POS_CHUNK)
    src_p = jnp.concatenate(
        [edge[0].astype(jnp.int32), jnp.arange(E_PAD - E, dtype=jnp.int32) % N]
    ).reshape(NW, CH_PER_W, CHUNK)
    dst_p = jnp.concatenate(
        [edge[1].astype(jnp.int32),
         N + jnp.arange(E_PAD - E, dtype=jnp.int32) % DUMP_ROWS]
    ).reshape(NW, CH_PER_W, CHUNK)
    zeros_blk = jnp.zeros((ROWS_PER_TILE, D), f32)
    g_e, b_e = emb_ln_g.reshape(1, D), emb_ln_b.reshape(1, D)

    _seg_sum = _seg_sum_kernel()
    pos_emb = _pos_gather_kernel()(pos_table, pos_p)
    h = _embed_call(node_emb, pos_emb, g_e, b_e)

    p = _seg_sum(h, src_p, dst_p, zeros_blk)
    h = _layer_call(p, h, Wl0, bl0.reshape(1, D), Wr0,
                    ln0_g.reshape(1, D), ln0_b.reshape(1, D))

    p = _seg_sum(h, src_p, dst_p, zeros_blk)
    h = _layer_call(p, h, Wl1, bl1.reshape(1, D), Wr1,
                    ln1_g.reshape(1, D), ln1_b.reshape(1, D))
    return h


# speedup vs baseline: 1.0228x; 1.0228x over previous
import functools
import math

import jax
import jax.numpy as jnp
from jax import lax
from jax.experimental import pallas as pl
from jax.experimental.pallas import tpu as pltpu
from jax.experimental.pallas import tpu_sc as plsc

N = 10000
E = 320000
D = 128
POS_VOC = 1024

NC, NS = 2, 16
NW = NC * NS

CHUNK = 64
CH_PER_W = 160
E_PER_W = CHUNK * CH_PER_W
E_PAD = NW * E_PER_W
ROWS_PAD = 10240
ROWS_PER_TILE = ROWS_PAD // NS
DUMP_ROWS = ROWS_PAD - N

NBUF = 5
SLACK = 2
NPHASE = 10
CH_PER_PH = CH_PER_W // NPHASE

POS_CHUNK = 64
POS_CH_PER_W = 5
POS_PER_W = POS_CHUNK * POS_CH_PER_W
POS_PAD = NW * POS_PER_W


@functools.cache
def _sc_mesh():
    return plsc.VectorSubcoreMesh(
        core_axis_name="c", subcore_axis_name="s", num_cores=NC, num_subcores=NS)


@functools.cache
def _pos_gather_kernel():
    @functools.partial(
        pl.kernel,
        out_type=jax.ShapeDtypeStruct((POS_PAD, D), jnp.float32),
        mesh=_sc_mesh(),
        scratch_types=[
            pltpu.VMEM((POS_CH_PER_W, POS_CHUNK), jnp.int32),
            pltpu.VMEM((POS_CHUNK, D), jnp.float32),
            pltpu.SemaphoreType.DMA,
        ],
    )
    def _pos_gather(table_hbm, pos_hbm, out_hbm, idx_v, rows_v, sem):
        wid = lax.axis_index("c") * NS + lax.axis_index("s")
        pltpu.sync_copy(pos_hbm.at[wid], idx_v)
        for j in range(POS_CH_PER_W):
            pltpu.async_copy(table_hbm.at[idx_v.at[j]], rows_v, sem).wait()
            pltpu.sync_copy(
                rows_v, out_hbm.at[pl.ds(wid * POS_PER_W + j * POS_CHUNK, POS_CHUNK)])

    return _pos_gather


@functools.cache
def _seg_sum_kernel():
    @functools.partial(
        pl.kernel,
        out_type=jax.ShapeDtypeStruct((NC, ROWS_PAD, D), jnp.float32),
        mesh=_sc_mesh(),
        scratch_types=[
            pltpu.VMEM((CH_PER_PH, CHUNK), jnp.int32),
            pltpu.VMEM((CH_PER_PH, CHUNK), jnp.int32),
            pltpu.VMEM((NBUF, CHUNK, D), jnp.float32),
            pltpu.VMEM_SHARED((ROWS_PAD, D), jnp.float32),
        ] + [pltpu.SemaphoreType.DMA] * (2 * NBUF),
    )
    def _seg_sum(h_hbm, src_hbm, dst_hbm, zeros_hbm, out_hbm,
                 idx_s_v, idx_d_v, rows_v, acc_sh, *sems):
        gsems, ssems = sems[:NBUF], sems[NBUF:]
        cid = lax.axis_index("c")
        sid = lax.axis_index("s")
        wid = cid * NS + sid

        def start_gather(b, cj):
            pltpu.async_copy(h_hbm.at[idx_s_v.at[cj]], rows_v.at[b], gsems[b])

        def wait_gather(b):
            pltpu.make_async_copy(
                h_hbm.at[idx_s_v.at[0]], rows_v.at[b], gsems[b]).wait()

        def start_scatter(b, cj):
            pltpu.async_copy(rows_v.at[b], acc_sh.at[idx_d_v.at[cj]],
                             ssems[b], add=True)

        def wait_scatter(b):
            pltpu.make_async_copy(
                rows_v.at[b], acc_sh.at[idx_d_v.at[0]], ssems[b]).wait()

        for ph in range(NPHASE):
            pltpu.sync_copy(src_hbm.at[wid, pl.ds(ph * CH_PER_PH, CH_PER_PH)], idx_s_v)
            pltpu.sync_copy(dst_hbm.at[wid, pl.ds(ph * CH_PER_PH, CH_PER_PH)], idx_d_v)

            for b in range(NBUF):
                start_gather(b, b)
            if ph == 0:
                pltpu.sync_copy(
                    zeros_hbm, acc_sh.at[pl.ds(sid * ROWS_PER_TILE, ROWS_PER_TILE)])
                plsc.subcore_barrier()
            for c in range(CH_PER_PH):
                wait_gather(c % NBUF)
                start_scatter(c % NBUF, c)
                if c >= SLACK and c - SLACK + NBUF < CH_PER_PH:
                    wait_scatter((c - SLACK) % NBUF)
                    start_gather((c - SLACK) % NBUF, c - SLACK + NBUF)
            for c in range(CH_PER_PH - NBUF, CH_PER_PH):
                wait_scatter(c % NBUF)
        plsc.subcore_barrier()
        pltpu.sync_copy(acc_sh.at[pl.ds(sid * ROWS_PER_TILE, ROWS_PER_TILE)],
                        out_hbm.at[cid, pl.ds(sid * ROWS_PER_TILE, ROWS_PER_TILE)])

    return _seg_sum


def _ln(x, g, b):
    m = jnp.mean(x, axis=1, keepdims=True)
    v = jnp.mean((x - m) ** 2, axis=1, keepdims=True)
    return (x - m) * lax.rsqrt(v + 1e-5) * g + b


def _embed_body(ne_ref, pe_ref, g_ref, b_ref, o_ref):
    x = ne_ref[...] * math.sqrt(D) + pe_ref[...]
    o_ref[...] = _ln(x, g_ref[...], b_ref[...])


def _matT(x, w_ref):
    return lax.dot_general(x, w_ref[...], (((1,), (1,)), ((), ())),
                           preferred_element_type=jnp.float32)


def _layer_body(p_ref, h_ref, wl_ref, bl_ref, wr_ref, g_ref, b_ref, o_ref):
    h = h_ref[...]
    agg = p_ref[0] + p_ref[1]
    z = _matT(agg, wl_ref) + bl_ref[...] + _matT(h, wr_ref)
    z = jnp.maximum(z, 0.0) + h
    o_ref[...] = _ln(z, g_ref[...], b_ref[...])


_ROW_BLK = 2000


def _row_spec():
    return pl.BlockSpec((_ROW_BLK, D), lambda i: (i, 0))


def _full_spec(r):
    return pl.BlockSpec((r, D), lambda i: (0, 0))


_embed_call = pl.pallas_call(
    _embed_body,
    grid=(N // _ROW_BLK,),
    in_specs=[_row_spec(), _row_spec(), _full_spec(1), _full_spec(1)],
    out_specs=_row_spec(),
    out_shape=jax.ShapeDtypeStruct((N, D), jnp.float32),
)

_layer_call = pl.pallas_call(
    _layer_body,
    grid=(N // _ROW_BLK,),
    in_specs=[pl.BlockSpec((NC, _ROW_BLK, D), lambda i: (0, i, 0)),
              _row_spec(),
              _full_spec(D), _full_spec(1), _full_spec(D),
              _full_spec(1), _full_spec(1)],
    out_specs=_row_spec(),
    out_shape=jax.ShapeDtypeStruct((N, D), jnp.float32),
)


def kernel(node_emb, pos, edge, pos_table, emb_ln_g, emb_ln_b,
           Wl0, bl0, Wr0, ln0_g, ln0_b, Wl1, bl1, Wr1, ln1_g, ln1_b):
    f32 = jnp.float32
    pos_p = jnp.concatenate(
        [pos.astype(jnp.int32), jnp.arange(POS_PAD - N, dtype=jnp.int32) % POS_VOC]
    ).reshape(NW, POS_CH_PER_W, POS_CHUNK)
    src_p = jnp.concatenate(
        [edge[0].astype(jnp.int32), jnp.arange(E_PAD - E, dtype=jnp.int32) % N]
    ).reshape(NW, CH_PER_W, CHUNK)
    dst_p = jnp.concatenate(
        [edge[1].astype(jnp.int32),
         N + jnp.arange(E_PAD - E, dtype=jnp.int32) % DUMP_ROWS]
    ).reshape(NW, CH_PER_W, CHUNK)
    zeros_blk = jnp.zeros((ROWS_PER_TILE, D), f32)
    g_e, b_e = emb_ln_g.reshape(1, D), emb_ln_b.reshape(1, D)

    _seg_sum = _seg_sum_kernel()
    pos_emb = _pos_gather_kernel()(pos_table, pos_p)
    h = _embed_call(node_emb, pos_emb, g_e, b_e)

    p = _seg_sum(h, src_p, dst_p, zeros_blk)
    h = _layer_call(p, h, Wl0, bl0.reshape(1, D), Wr0,
                    ln0_g.reshape(1, D), ln0_b.reshape(1, D))

    p = _seg_sum(h, src_p, dst_p, zeros_blk)
    h = _layer_call(p, h, Wl1, bl1.reshape(1, D), Wr1,
                    ln1_g.reshape(1, D), ln1_b.reshape(1, D))
    return h


# speedup vs baseline: 1.0438x; 1.0205x over previous
import functools
import math

import jax
import jax.numpy as jnp
from jax import lax
from jax.experimental import pallas as pl
from jax.experimental.pallas import tpu as pltpu
from jax.experimental.pallas import tpu_sc as plsc

N = 10000
E = 320000
D = 128
POS_VOC = 1024

NC, NS = 2, 16
NW = NC * NS

CHUNK = 64
CH_PER_W = 160
E_PER_W = CHUNK * CH_PER_W
E_PAD = NW * E_PER_W
ROWS_PAD = 10240
ROWS_PER_TILE = ROWS_PAD // NS
DUMP_ROWS = ROWS_PAD - N

NBUF = 5
SLACK = 1
NPHASE = 10
CH_PER_PH = CH_PER_W // NPHASE

POS_CHUNK = 64
POS_CH_PER_W = 5
POS_PER_W = POS_CHUNK * POS_CH_PER_W
POS_PAD = NW * POS_PER_W


@functools.cache
def _sc_mesh():
    return plsc.VectorSubcoreMesh(
        core_axis_name="c", subcore_axis_name="s", num_cores=NC, num_subcores=NS)


@functools.cache
def _pos_gather_kernel():
    @functools.partial(
        pl.kernel,
        out_type=jax.ShapeDtypeStruct((POS_PAD, D), jnp.float32),
        mesh=_sc_mesh(),
        scratch_types=[
            pltpu.VMEM((POS_CH_PER_W, POS_CHUNK), jnp.int32),
            pltpu.VMEM((POS_CHUNK, D), jnp.float32),
            pltpu.SemaphoreType.DMA,
        ],
    )
    def _pos_gather(table_hbm, pos_hbm, out_hbm, idx_v, rows_v, sem):
        wid = lax.axis_index("c") * NS + lax.axis_index("s")
        pltpu.sync_copy(pos_hbm.at[wid], idx_v)
        for j in range(POS_CH_PER_W):
            pltpu.async_copy(table_hbm.at[idx_v.at[j]], rows_v, sem).wait()
            pltpu.sync_copy(
                rows_v, out_hbm.at[pl.ds(wid * POS_PER_W + j * POS_CHUNK, POS_CHUNK)])

    return _pos_gather


@functools.cache
def _seg_sum_kernel():
    @functools.partial(
        pl.kernel,
        out_type=jax.ShapeDtypeStruct((NC, ROWS_PAD, D), jnp.float32),
        mesh=_sc_mesh(),
        scratch_types=[
            pltpu.VMEM((CH_PER_PH, CHUNK), jnp.int32),
            pltpu.VMEM((CH_PER_PH, CHUNK), jnp.int32),
            pltpu.VMEM((NBUF, CHUNK, D), jnp.float32),
            pltpu.VMEM_SHARED((ROWS_PAD, D), jnp.float32),
        ] + [pltpu.SemaphoreType.DMA] * (2 * NBUF),
    )
    def _seg_sum(h_hbm, src_hbm, dst_hbm, zeros_hbm, out_hbm,
                 idx_s_v, idx_d_v, rows_v, acc_sh, *sems):
        gsems, ssems = sems[:NBUF], sems[NBUF:]
        cid = lax.axis_index("c")
        sid = lax.axis_index("s")
        wid = cid * NS + sid

        def start_gather(b, cj):
            pltpu.async_copy(h_hbm.at[idx_s_v.at[cj]], rows_v.at[b], gsems[b])

        def wait_gather(b):
            pltpu.make_async_copy(
                h_hbm.at[idx_s_v.at[0]], rows_v.at[b], gsems[b]).wait()

        def start_scatter(b, cj):
            pltpu.async_copy(rows_v.at[b], acc_sh.at[idx_d_v.at[cj]],
                             ssems[b], add=True)

        def wait_scatter(b):
            pltpu.make_async_copy(
                rows_v.at[b], acc_sh.at[idx_d_v.at[0]], ssems[b]).wait()

        for ph in range(NPHASE):
            pltpu.sync_copy(src_hbm.at[wid, pl.ds(ph * CH_PER_PH, CH_PER_PH)], idx_s_v)
            pltpu.sync_copy(dst_hbm.at[wid, pl.ds(ph * CH_PER_PH, CH_PER_PH)], idx_d_v)

            for b in range(NBUF):
                start_gather(b, b)
            if ph == 0:
                pltpu.sync_copy(
                    zeros_hbm, acc_sh.at[pl.ds(sid * ROWS_PER_TILE, ROWS_PER_TILE)])
                plsc.subcore_barrier()
            for c in range(CH_PER_PH):
                wait_gather(c % NBUF)
                start_scatter(c % NBUF, c)
                if c >= SLACK and c - SLACK + NBUF < CH_PER_PH:
                    wait_scatter((c - SLACK) % NBUF)
                    start_gather((c - SLACK) % NBUF, c - SLACK + NBUF)
            for c in range(CH_PER_PH - NBUF, CH_PER_PH):
                wait_scatter(c % NBUF)
        plsc.subcore_barrier()
        pltpu.sync_copy(acc_sh.at[pl.ds(sid * ROWS_PER_TILE, ROWS_PER_TILE)],
                        out_hbm.at[cid, pl.ds(sid * ROWS_PER_TILE, ROWS_PER_TILE)])

    return _seg_sum


def _ln(x, g, b):
    m = jnp.mean(x, axis=1, keepdims=True)
    v = jnp.mean((x - m) ** 2, axis=1, keepdims=True)
    return (x - m) * lax.rsqrt(v + 1e-5) * g + b


def _embed_body(ne_ref, pe_ref, g_ref, b_ref, o_ref):
    x = ne_ref[...] * math.sqrt(D) + pe_ref[...]
    o_ref[...] = _ln(x, g_ref[...], b_ref[...])


def _matT(x, w_ref):
    return lax.dot_general(x, w_ref[...], (((1,), (1,)), ((), ())),
                           preferred_element_type=jnp.float32)


def _layer_body(p_ref, h_ref, wl_ref, bl_ref, wr_ref, g_ref, b_ref, o_ref):
    h = h_ref[...]
    agg = p_ref[0] + p_ref[1]
    z = _matT(agg, wl_ref) + bl_ref[...] + _matT(h, wr_ref)
    z = jnp.maximum(z, 0.0) + h
    o_ref[...] = _ln(z, g_ref[...], b_ref[...])


_ROW_BLK = 2000


def _row_spec():
    return pl.BlockSpec((_ROW_BLK, D), lambda i: (i, 0))


def _full_spec(r):
    return pl.BlockSpec((r, D), lambda i: (0, 0))


_embed_call = pl.pallas_call(
    _embed_body,
    grid=(N // _ROW_BLK,),
    in_specs=[_row_spec(), _row_spec(), _full_spec(1), _full_spec(1)],
    out_specs=_row_spec(),
    out_shape=jax.ShapeDtypeStruct((N, D), jnp.float32),
)

_layer_call = pl.pallas_call(
    _layer_body,
    grid=(N // _ROW_BLK,),
    in_specs=[pl.BlockSpec((NC, _ROW_BLK, D), lambda i: (0, i, 0)),
              _row_spec(),
              _full_spec(D), _full_spec(1), _full_spec(D),
              _full_spec(1), _full_spec(1)],
    out_specs=_row_spec(),
    out_shape=jax.ShapeDtypeStruct((N, D), jnp.float32),
)


def kernel(node_emb, pos, edge, pos_table, emb_ln_g, emb_ln_b,
           Wl0, bl0, Wr0, ln0_g, ln0_b, Wl1, bl1, Wr1, ln1_g, ln1_b):
    f32 = jnp.float32
    pos_p = jnp.concatenate(
        [pos.astype(jnp.int32), jnp.arange(POS_PAD - N, dtype=jnp.int32) % POS_VOC]
    ).reshape(NW, POS_CH_PER_W, POS_CHUNK)
    src_p = jnp.concatenate(
        [edge[0].astype(jnp.int32), jnp.arange(E_PAD - E, dtype=jnp.int32) % N]
    ).reshape(NW, CH_PER_W, CHUNK)
    dst_p = jnp.concatenate(
        [edge[1].astype(jnp.int32),
         N + jnp.arange(E_PAD - E, dtype=jnp.int32) % DUMP_ROWS]
    ).reshape(NW, CH_PER_W, CHUNK)
    zeros_blk = jnp.zeros((ROWS_PER_TILE, D), f32)
    g_e, b_e = emb_ln_g.reshape(1, D), emb_ln_b.reshape(1, D)

    _seg_sum = _seg_sum_kernel()
    pos_emb = _pos_gather_kernel()(pos_table, pos_p)
    h = _embed_call(node_emb, pos_emb, g_e, b_e)

    p = _seg_sum(h, src_p, dst_p, zeros_blk)
    h = _layer_call(p, h, Wl0, bl0.reshape(1, D), Wr0,
                    ln0_g.reshape(1, D), ln0_b.reshape(1, D))

    p = _seg_sum(h, src_p, dst_p, zeros_blk)
    h = _layer_call(p, h, Wl1, bl1.reshape(1, D), Wr1,
                    ln1_g.reshape(1, D), ln1_b.reshape(1, D))
    return h


# speedup vs baseline: 1.0670x; 1.0222x over previous
import functools
import math

import jax
import jax.numpy as jnp
from jax import lax
from jax.experimental import pallas as pl
from jax.experimental.pallas import tpu as pltpu
from jax.experimental.pallas import tpu_sc as plsc

N = 10000
E = 320000
D = 128
POS_VOC = 1024

NC, NS = 2, 16
NW = NC * NS

CHUNK = 64
CH_PER_W = 160
E_PER_W = CHUNK * CH_PER_W
E_PAD = NW * E_PER_W
ROWS_PAD = 10240
ROWS_PER_TILE = ROWS_PAD // NS
DUMP_ROWS = ROWS_PAD - N

NBUF = 5
SLACK = 1
NPHASE = 10
CH_PER_PH = CH_PER_W // NPHASE

POS_CHUNK = 64
POS_CH_PER_W = 5
POS_PER_W = POS_CHUNK * POS_CH_PER_W
POS_PAD = NW * POS_PER_W


@functools.cache
def _sc_mesh():
    return plsc.VectorSubcoreMesh(
        core_axis_name="c", subcore_axis_name="s", num_cores=NC, num_subcores=NS)


@functools.cache
def _pos_gather_kernel():
    @functools.partial(
        pl.kernel,
        out_type=jax.ShapeDtypeStruct((POS_PAD, D), jnp.float32),
        mesh=_sc_mesh(),
        scratch_types=[
            pltpu.VMEM((POS_CH_PER_W, POS_CHUNK), jnp.int32),
            pltpu.VMEM((POS_CHUNK, D), jnp.float32),
            pltpu.SemaphoreType.DMA,
        ],
    )
    def _pos_gather(table_hbm, pos_hbm, out_hbm, idx_v, rows_v, sem):
        wid = lax.axis_index("c") * NS + lax.axis_index("s")
        pltpu.sync_copy(pos_hbm.at[wid], idx_v)
        for j in range(POS_CH_PER_W):
            pltpu.async_copy(table_hbm.at[idx_v.at[j]], rows_v, sem).wait()
            pltpu.sync_copy(
                rows_v, out_hbm.at[pl.ds(wid * POS_PER_W + j * POS_CHUNK, POS_CHUNK)])

    return _pos_gather


@functools.cache
def _seg_sum_kernel():
    @functools.partial(
        pl.kernel,
        out_type=jax.ShapeDtypeStruct((NC, ROWS_PAD, D), jnp.float32),
        mesh=_sc_mesh(),
        scratch_types=[
            pltpu.VMEM((CH_PER_PH, CHUNK), jnp.int32),
            pltpu.VMEM((CH_PER_PH, CHUNK), jnp.int32),
            pltpu.VMEM((NBUF, CHUNK, D), jnp.float32),
            pltpu.VMEM_SHARED((ROWS_PAD, D), jnp.float32),
        ] + [pltpu.SemaphoreType.DMA] * (3 * NBUF),
    )
    def _seg_sum(h_hbm, src_hbm, dst_hbm, zeros_hbm, out_hbm,
                 idx_s_v, idx_d_v, rows_v, acc_sh, *sems):
        gsems, gsems2, ssems = sems[:NBUF], sems[NBUF:2 * NBUF], sems[2 * NBUF:]
        cid = lax.axis_index("c")
        sid = lax.axis_index("s")
        wid = cid * NS + sid
        HC = CHUNK // 2

        def start_gather(b, cj):
            pltpu.async_copy(h_hbm.at[idx_s_v.at[cj, pl.ds(0, HC)]],
                             rows_v.at[b, pl.ds(0, HC)], gsems[b])
            pltpu.async_copy(h_hbm.at[idx_s_v.at[cj, pl.ds(HC, HC)]],
                             rows_v.at[b, pl.ds(HC, HC)], gsems2[b])

        def wait_gather(b):
            pltpu.make_async_copy(
                h_hbm.at[idx_s_v.at[0, pl.ds(0, HC)]],
                rows_v.at[b, pl.ds(0, HC)], gsems[b]).wait()
            pltpu.make_async_copy(
                h_hbm.at[idx_s_v.at[0, pl.ds(HC, HC)]],
                rows_v.at[b, pl.ds(HC, HC)], gsems2[b]).wait()

        def start_scatter(b, cj):
            pltpu.async_copy(rows_v.at[b], acc_sh.at[idx_d_v.at[cj]],
                             ssems[b], add=True)

        def wait_scatter(b):
            pltpu.make_async_copy(
                rows_v.at[b], acc_sh.at[idx_d_v.at[0]], ssems[b]).wait()

        for ph in range(NPHASE):
            pltpu.sync_copy(src_hbm.at[wid, pl.ds(ph * CH_PER_PH, CH_PER_PH)], idx_s_v)
            pltpu.sync_copy(dst_hbm.at[wid, pl.ds(ph * CH_PER_PH, CH_PER_PH)], idx_d_v)

            for b in range(NBUF):
                start_gather(b, b)
            if ph == 0:
                pltpu.sync_copy(
                    zeros_hbm, acc_sh.at[pl.ds(sid * ROWS_PER_TILE, ROWS_PER_TILE)])
                plsc.subcore_barrier()
            for c in range(CH_PER_PH):
                wait_gather(c % NBUF)
                start_scatter(c % NBUF, c)
                if c >= SLACK and c - SLACK + NBUF < CH_PER_PH:
                    wait_scatter((c - SLACK) % NBUF)
                    start_gather((c - SLACK) % NBUF, c - SLACK + NBUF)
            for c in range(CH_PER_PH - NBUF, CH_PER_PH):
                wait_scatter(c % NBUF)
        plsc.subcore_barrier()
        pltpu.sync_copy(acc_sh.at[pl.ds(sid * ROWS_PER_TILE, ROWS_PER_TILE)],
                        out_hbm.at[cid, pl.ds(sid * ROWS_PER_TILE, ROWS_PER_TILE)])

    return _seg_sum


def _ln(x, g, b):
    m = jnp.mean(x, axis=1, keepdims=True)
    v = jnp.mean((x - m) ** 2, axis=1, keepdims=True)
    return (x - m) * lax.rsqrt(v + 1e-5) * g + b


def _embed_body(ne_ref, pe_ref, g_ref, b_ref, o_ref):
    x = ne_ref[...] * math.sqrt(D) + pe_ref[...]
    o_ref[...] = _ln(x, g_ref[...], b_ref[...])


def _matT(x, w_ref):
    return lax.dot_general(x, w_ref[...], (((1,), (1,)), ((), ())),
                           preferred_element_type=jnp.float32)


def _layer_body(p_ref, h_ref, wl_ref, bl_ref, wr_ref, g_ref, b_ref, o_ref):
    h = h_ref[...]
    agg = p_ref[0] + p_ref[1]
    z = _matT(agg, wl_ref) + bl_ref[...] + _matT(h, wr_ref)
    z = jnp.maximum(z, 0.0) + h
    o_ref[...] = _ln(z, g_ref[...], b_ref[...])


_ROW_BLK = 2000


def _row_spec():
    return pl.BlockSpec((_ROW_BLK, D), lambda i: (i, 0))


def _full_spec(r):
    return pl.BlockSpec((r, D), lambda i: (0, 0))


_embed_call = pl.pallas_call(
    _embed_body,
    grid=(N // _ROW_BLK,),
    in_specs=[_row_spec(), _row_spec(), _full_spec(1), _full_spec(1)],
    out_specs=_row_spec(),
    out_shape=jax.ShapeDtypeStruct((N, D), jnp.float32),
)

_layer_call = pl.pallas_call(
    _layer_body,
    grid=(N // _ROW_BLK,),
    in_specs=[pl.BlockSpec((NC, _ROW_BLK, D), lambda i: (0, i, 0)),
              _row_spec(),
              _full_spec(D), _full_spec(1), _full_spec(D),
              _full_spec(1), _full_spec(1)],
    out_specs=_row_spec(),
    out_shape=jax.ShapeDtypeStruct((N, D), jnp.float32),
)


def kernel(node_emb, pos, edge, pos_table, emb_ln_g, emb_ln_b,
           Wl0, bl0, Wr0, ln0_g, ln0_b, Wl1, bl1, Wr1, ln1_g, ln1_b):
    f32 = jnp.float32
    pos_p = jnp.concatenate(
        [pos.astype(jnp.int32), jnp.arange(POS_PAD - N, dtype=jnp.int32) % POS_VOC]
    ).reshape(NW, POS_CH_PER_W, POS_CHUNK)
    src_p = jnp.concatenate(
        [edge[0].astype(jnp.int32), jnp.arange(E_PAD - E, dtype=jnp.int32) % N]
    ).reshape(NW, CH_PER_W, CHUNK)
    dst_p = jnp.concatenate(
        [edge[1].astype(jnp.int32),
         N + jnp.arange(E_PAD - E, dtype=jnp.int32) % DUMP_ROWS]
    ).reshape(NW, CH_PER_W, CHUNK)
    zeros_blk = jnp.zeros((ROWS_PER_TILE, D), f32)
    g_e, b_e = emb_ln_g.reshape(1, D), emb_ln_b.reshape(1, D)

    _seg_sum = _seg_sum_kernel()
    pos_emb = _pos_gather_kernel()(pos_table, pos_p)
    h = _embed_call(node_emb, pos_emb, g_e, b_e)

    p = _seg_sum(h, src_p, dst_p, zeros_blk)
    h = _layer_call(p, h, Wl0, bl0.reshape(1, D), Wr0,
                    ln0_g.reshape(1, D), ln0_b.reshape(1, D))

    p = _seg_sum(h, src_p, dst_p, zeros_blk)
    h = _layer_call(p, h, Wl1, bl1.reshape(1, D), Wr1,
                    ln1_g.reshape(1, D), ln1_b.reshape(1, D))
    return h
